# baseline (speedup 1.0000x reference)
import functools

import jax
import jax.numpy as jnp
from jax import lax
from jax.experimental import pallas as pl
from jax.experimental.pallas import tpu as pltpu
from jax.experimental.pallas import tpu_sc as plsc

N_NODES_C = 100000
N_EDGES_C = 6400000
_EPS = 1.0
_SIG = 1.0
_CUT = 5.0
_HALF_SHIFT = 2.0 * _EPS * ((_SIG / _CUT) ** 12 - (_SIG / _CUT) ** 6)

NC = 2
NS = 16
NW = NC * NS
EPW = N_EDGES_C // NW
CHUNK = 800
NCHUNK = EPW // CHUNK
NPAIR = NCHUNK // 2
GROUPS = CHUNK // 16
UNROLL = 5

ROWS = 112
COLS = 1024


def _sc_body(x_hbm, y_hbm, z_hbm, i_hbm, j_hbm, out_hbm,
             acc, x0, y0, z0, i0, j0, x1, y1, z1, i1, j1, sem0, sem1):
    cid = lax.axis_index("c")
    sid = lax.axis_index("s")
    wid = cid * NS + sid
    ebase = wid * EPW

    slots = ((x0, y0, z0, i0, j0, sem0), (x1, y1, z1, i1, j1, sem1))
    hbms = (x_hbm, y_hbm, z_hbm, i_hbm, j_hbm)

    def issue5(slot, k):
        base = ebase + k * CHUNK
        for hbm, buf in zip(hbms, slot[:5]):
            pltpu.async_copy(hbm.at[pl.ds(base, CHUNK)], buf, slot[5])

    def wait5(slot):
        for hbm, buf in zip(hbms, slot[:5]):
            pltpu.make_async_copy(hbm.at[pl.ds(0, CHUNK)], buf, slot[5]).wait()

    issue5(slots[0], 0)

    zero16 = jnp.zeros((16,), jnp.float32)

    def _zero_row(r, _):
        def _zero_col(t, _):
            for q in range(4):
                acc[r, pl.ds(t * 64 + q * 16, 16)] = zero16
            return 0
        return lax.fori_loop(0, COLS // 64, _zero_col, 0)

    lax.fori_loop(0, ROWS, _zero_row, 0)

    def compute(slot):
        xb, yb, zb, ib, jb = slot[:5]

        def _group(g, _):
            for u in range(UNROLL):
                o = (g * UNROLL + u) * 16
                dx = xb[pl.ds(o, 16)]
                dy = yb[pl.ds(o, 16)]
                dz = zb[pl.ds(o, 16)]
                r2 = dx * dx + dy * dy + dz * dz
                seed = plsc.bitcast(
                    jnp.int32(0x7EF311C3) - plsc.bitcast(r2, jnp.int32),
                    jnp.float32)
                inv = seed * (2.0 - r2 * seed)
                inv = inv * (2.0 - r2 * inv)
                inv = inv * (2.0 - r2 * inv)
                s6 = inv * inv * inv
                he = 2.0 * _EPS * (s6 * s6 - s6) - _HALF_SHIFT
                iv = ib[pl.ds(o, 16)]
                jv = jb[pl.ds(o, 16)]
                plsc.addupdate_scatter(
                    acc, [lax.shift_right_logical(iv, 10),
                          lax.bitwise_and(iv, 1023)], he)
                plsc.addupdate_scatter(
                    acc, [lax.shift_right_logical(jv, 10),
                          lax.bitwise_and(jv, 1023)], he)
            return 0

        lax.fori_loop(0, GROUPS // UNROLL, _group, 0)

    def _pair(t, _):
        issue5(slots[1], 2 * t + 1)
        wait5(slots[0])
        compute(slots[0])

        @pl.when(t < NPAIR - 1)
        def _():
            issue5(slots[0], 2 * t + 2)

        wait5(slots[1])
        compute(slots[1])
        return 0

    lax.fori_loop(0, NPAIR, _pair, 0)

    pltpu.sync_copy(acc, out_hbm.at[wid])


@functools.partial(jax.jit, static_argnames=())
def _sc_partials(xs, ys, zs, all_i, all_j):
    mesh = plsc.VectorSubcoreMesh(core_axis_name="c", subcore_axis_name="s",
                                  num_cores=NC, num_subcores=NS)
    return pl.kernel(
        _sc_body,
        out_type=jax.ShapeDtypeStruct((NW, ROWS, COLS), jnp.float32),
        mesh=mesh,
        compiler_params=pltpu.CompilerParams(needs_layout_passes=False,
                                             use_tc_tiling_on_sc=False),
        scratch_types=[
            pltpu.VMEM((ROWS, COLS), jnp.float32),
            pltpu.VMEM((CHUNK,), jnp.float32),
            pltpu.VMEM((CHUNK,), jnp.float32),
            pltpu.VMEM((CHUNK,), jnp.float32),
            pltpu.VMEM((CHUNK,), jnp.int32),
            pltpu.VMEM((CHUNK,), jnp.int32),
            pltpu.VMEM((CHUNK,), jnp.float32),
            pltpu.VMEM((CHUNK,), jnp.float32),
            pltpu.VMEM((CHUNK,), jnp.float32),
            pltpu.VMEM((CHUNK,), jnp.int32),
            pltpu.VMEM((CHUNK,), jnp.int32),
            pltpu.SemaphoreType.DMA,
            pltpu.SemaphoreType.DMA,
        ],
    )(xs, ys, zs, all_i, all_j)


def _sum_body(p_ref, b_ref, o_ref):
    o_ref[...] = jnp.sum(p_ref[...], axis=0) + b_ref[...]


def _tc_sum(partials, bias):
    return pl.pallas_call(
        _sum_body,
        out_shape=jax.ShapeDtypeStruct((ROWS * COLS,), jnp.float32),
    )(partials.reshape(NW, ROWS * COLS), bias)


def kernel(distances, all_i, all_j, n_nodes):
    xs = distances[:, 0]
    ys = distances[:, 1]
    zs = distances[:, 2]
    partials = _sc_partials(xs, ys, zs, all_i, all_j)
    bias = jnp.full((1,), 0.0, jnp.float32) + (
        jnp.asarray(n_nodes, jnp.float32) - float(N_NODES_C))
    summed = _tc_sum(partials, bias)
    return summed[:N_NODES_C].reshape(-1, 1)


# speedup vs baseline: 1.1278x; 1.1278x over previous
import functools

import jax
import jax.numpy as jnp
from jax import lax
from jax.experimental import pallas as pl
from jax.experimental.pallas import tpu as pltpu
from jax.experimental.pallas import tpu_sc as plsc

N_NODES_C = 100000
N_EDGES_C = 6400000
_EPS = 1.0
_SIG = 1.0
_CUT = 5.0
_HALF_SHIFT = 2.0 * _EPS * ((_SIG / _CUT) ** 12 - (_SIG / _CUT) ** 6)

NC = 2
NS = 16
NW = NC * NS
EPW = N_EDGES_C // NW
CHUNK = 800
NCHUNK = EPW // CHUNK
NPAIR = NCHUNK // 2
GROUPS = CHUNK // 16
UNROLL = 10

ROWS = 112
COLS = 1024


def _sc_body(x_hbm, y_hbm, z_hbm, i_hbm, j_hbm, out_hbm,
             acc, x0, y0, z0, i0, j0, x1, y1, z1, i1, j1, sem0, sem1):
    cid = lax.axis_index("c")
    sid = lax.axis_index("s")
    wid = cid * NS + sid
    ebase = wid * EPW

    slots = ((x0, y0, z0, i0, j0, sem0), (x1, y1, z1, i1, j1, sem1))
    hbms = (x_hbm, y_hbm, z_hbm, i_hbm, j_hbm)

    def issue5(slot, k):
        base = ebase + k * CHUNK
        for hbm, buf in zip(hbms, slot[:5]):
            pltpu.async_copy(hbm.at[pl.ds(base, CHUNK)], buf, slot[5])

    def wait5(slot):
        for hbm, buf in zip(hbms, slot[:5]):
            pltpu.make_async_copy(hbm.at[pl.ds(0, CHUNK)], buf, slot[5]).wait()

    issue5(slots[0], 0)

    zero16 = jnp.zeros((16,), jnp.float32)

    def _zero(t, _):
        for q in range(4):
            acc[pl.ds(t * 64 + q * 16, 16)] = zero16
        return 0

    lax.fori_loop(0, (ROWS * COLS) // 64, _zero, 0)

    def compute(slot):
        xb, yb, zb, ib, jb = slot[:5]

        def _group(g, _):
            for u in range(UNROLL):
                o = (g * UNROLL + u) * 16
                dx = xb[pl.ds(o, 16)]
                dy = yb[pl.ds(o, 16)]
                dz = zb[pl.ds(o, 16)]
                r2 = dx * dx + dy * dy + dz * dz
                inv = 1.0 / r2
                s6 = inv * inv * inv
                he = 2.0 * _EPS * (s6 * s6 - s6) - _HALF_SHIFT
                iv = ib[pl.ds(o, 16)]
                jv = jb[pl.ds(o, 16)]
                plsc.addupdate_scatter(acc, [iv], he)
                plsc.addupdate_scatter(acc, [jv], he)
            return 0

        lax.fori_loop(0, GROUPS // UNROLL, _group, 0)

    def _pair(t, _):
        issue5(slots[1], 2 * t + 1)
        wait5(slots[0])
        compute(slots[0])

        @pl.when(t < NPAIR - 1)
        def _():
            issue5(slots[0], 2 * t + 2)

        wait5(slots[1])
        compute(slots[1])
        return 0

    lax.fori_loop(0, NPAIR, _pair, 0)

    pltpu.sync_copy(acc, out_hbm.at[wid])


@functools.partial(jax.jit, static_argnames=())
def _sc_partials(xs, ys, zs, all_i, all_j):
    mesh = plsc.VectorSubcoreMesh(core_axis_name="c", subcore_axis_name="s",
                                  num_cores=NC, num_subcores=NS)
    return pl.kernel(
        _sc_body,
        out_type=jax.ShapeDtypeStruct((NW, ROWS * COLS), jnp.float32),
        mesh=mesh,
        compiler_params=pltpu.CompilerParams(needs_layout_passes=False,
                                             use_tc_tiling_on_sc=False),
        scratch_types=[
            pltpu.VMEM((ROWS * COLS,), jnp.float32),
            pltpu.VMEM((CHUNK,), jnp.float32),
            pltpu.VMEM((CHUNK,), jnp.float32),
            pltpu.VMEM((CHUNK,), jnp.float32),
            pltpu.VMEM((CHUNK,), jnp.int32),
            pltpu.VMEM((CHUNK,), jnp.int32),
            pltpu.VMEM((CHUNK,), jnp.float32),
            pltpu.VMEM((CHUNK,), jnp.float32),
            pltpu.VMEM((CHUNK,), jnp.float32),
            pltpu.VMEM((CHUNK,), jnp.int32),
            pltpu.VMEM((CHUNK,), jnp.int32),
            pltpu.SemaphoreType.DMA,
            pltpu.SemaphoreType.DMA,
        ],
    )(xs, ys, zs, all_i, all_j)


def _sum_body(p_ref, b_ref, o_ref):
    o_ref[...] = jnp.sum(p_ref[...], axis=0) + b_ref[...]


def _tc_sum(partials, bias):
    return pl.pallas_call(
        _sum_body,
        out_shape=jax.ShapeDtypeStruct((ROWS * COLS,), jnp.float32),
    )(partials, bias)


def kernel(distances, all_i, all_j, n_nodes):
    xs = distances[:, 0]
    ys = distances[:, 1]
    zs = distances[:, 2]
    partials = _sc_partials(xs, ys, zs, all_i, all_j)
    bias = jnp.full((1,), 0.0, jnp.float32) + (
        jnp.asarray(n_nodes, jnp.float32) - float(N_NODES_C))
    summed = _tc_sum(partials, bias)
    return summed[:N_NODES_C].reshape(-1, 1)


# speedup vs baseline: 1.1428x; 1.0134x over previous
import functools

import jax
import jax.numpy as jnp
from jax import lax
from jax.experimental import pallas as pl
from jax.experimental.pallas import tpu as pltpu
from jax.experimental.pallas import tpu_sc as plsc

N_NODES_C = 100000
N_EDGES_C = 6400000
_EPS = 1.0
_SIG = 1.0
_CUT = 5.0
_HALF_SHIFT = 2.0 * _EPS * ((_SIG / _CUT) ** 12 - (_SIG / _CUT) ** 6)

NC = 2
NS = 16
NW = NC * NS
EPW = N_EDGES_C // NW
CHUNK = 2000
NCHUNK = EPW // CHUNK
NPAIR = NCHUNK // 2
GROUPS = CHUNK // 16
UNROLL = 5

ACC = 100352


def _sc_body(x_hbm, y_hbm, z_hbm, i_hbm, j_hbm, out_hbm,
             acc, x0, y0, z0, i0, j0, x1, y1, z1, i1, j1, sem0, sem1):
    cid = lax.axis_index("c")
    sid = lax.axis_index("s")
    wid = cid * NS + sid
    ebase = wid * EPW

    slots = ((x0, y0, z0, i0, j0, sem0), (x1, y1, z1, i1, j1, sem1))
    hbms = (x_hbm, y_hbm, z_hbm, i_hbm, j_hbm)

    def issue5(slot, k):
        base = ebase + k * CHUNK
        for hbm, buf in zip(hbms, slot[:5]):
            pltpu.async_copy(hbm.at[pl.ds(base, CHUNK)], buf, slot[5])

    def wait5(slot):
        for hbm, buf in zip(hbms, slot[:5]):
            pltpu.make_async_copy(hbm.at[pl.ds(0, CHUNK)], buf, slot[5]).wait()

    issue5(slots[0], 0)

    zero16 = jnp.zeros((16,), jnp.float32)

    def _zero(t, _):
        for q in range(4):
            acc[pl.ds(t * 64 + q * 16, 16)] = zero16
        return 0

    lax.fori_loop(0, ACC // 64, _zero, 0)

    def compute(slot):
        xb, yb, zb, ib, jb = slot[:5]

        def _group(g, _):
            for u in range(UNROLL):
                o = (g * UNROLL + u) * 16
                dx = xb[pl.ds(o, 16)]
                dy = yb[pl.ds(o, 16)]
                dz = zb[pl.ds(o, 16)]
                r2 = dx * dx + dy * dy + dz * dz
                inv = 1.0 / r2
                s6 = inv * inv * inv
                he = 2.0 * _EPS * (s6 * s6 - s6) - _HALF_SHIFT
                iv = ib[pl.ds(o, 16)]
                jv = jb[pl.ds(o, 16)]
                plsc.addupdate_scatter(acc, [iv], he)
                plsc.addupdate_scatter(acc, [jv], he)
            return 0

        lax.fori_loop(0, GROUPS // UNROLL, _group, 0)

    def _pair(t, _):
        issue5(slots[1], 2 * t + 1)
        wait5(slots[0])
        compute(slots[0])

        @pl.when(t < NPAIR - 1)
        def _():
            issue5(slots[0], 2 * t + 2)

        wait5(slots[1])
        compute(slots[1])
        return 0

    lax.fori_loop(0, NPAIR, _pair, 0)

    pltpu.sync_copy(acc, out_hbm.at[wid])


@functools.partial(jax.jit, static_argnames=())
def _sc_partials(xs, ys, zs, all_i, all_j):
    mesh = plsc.VectorSubcoreMesh(core_axis_name="c", subcore_axis_name="s",
                                  num_cores=NC, num_subcores=NS)
    return pl.kernel(
        _sc_body,
        out_type=jax.ShapeDtypeStruct((NW, ACC), jnp.float32),
        mesh=mesh,
        compiler_params=pltpu.CompilerParams(needs_layout_passes=False,
                                             use_tc_tiling_on_sc=False),
        scratch_types=[
            pltpu.VMEM((ACC,), jnp.float32),
            pltpu.VMEM((CHUNK,), jnp.float32),
            pltpu.VMEM((CHUNK,), jnp.float32),
            pltpu.VMEM((CHUNK,), jnp.float32),
            pltpu.VMEM((CHUNK,), jnp.int32),
            pltpu.VMEM((CHUNK,), jnp.int32),
            pltpu.VMEM((CHUNK,), jnp.float32),
            pltpu.VMEM((CHUNK,), jnp.float32),
            pltpu.VMEM((CHUNK,), jnp.float32),
            pltpu.VMEM((CHUNK,), jnp.int32),
            pltpu.VMEM((CHUNK,), jnp.int32),
            pltpu.SemaphoreType.DMA,
            pltpu.SemaphoreType.DMA,
        ],
    )(xs, ys, zs, all_i, all_j)


def _sum_body(p_ref, b_ref, o_ref):
    o_ref[...] = jnp.sum(p_ref[...], axis=0) + b_ref[...]


def _tc_sum(partials, bias):
    return pl.pallas_call(
        _sum_body,
        out_shape=jax.ShapeDtypeStruct((ACC,), jnp.float32),
    )(partials, bias)


def kernel(distances, all_i, all_j, n_nodes):
    xs = distances[:, 0]
    ys = distances[:, 1]
    zs = distances[:, 2]
    partials = _sc_partials(xs, ys, zs, all_i, all_j)
    bias = jnp.full((1,), 0.0, jnp.float32) + (
        jnp.asarray(n_nodes, jnp.float32) - float(N_NODES_C))
    summed = _tc_sum(partials, bias)
    return summed[:N_NODES_C].reshape(-1, 1)


# speedup vs baseline: 2.0178x; 1.7656x over previous
import functools

import jax
import jax.numpy as jnp
from jax import lax
from jax.experimental import pallas as pl
from jax.experimental.pallas import tpu as pltpu
from jax.experimental.pallas import tpu_sc as plsc

N_NODES_C = 100000
N_EDGES_C = 6400000
_EPS = 1.0
_SIG = 1.0
_CUT = 5.0
_HALF_SHIFT = 2.0 * _EPS * ((_SIG / _CUT) ** 12 - (_SIG / _CUT) ** 6)

NC = 2
NS = 16
NW = NC * NS
EPW = N_EDGES_C // NW
CHUNK = 2000
NCHUNK = EPW // CHUNK
NPAIR = NCHUNK // 2
GROUPS = CHUNK // 16
UNROLL = 5

ACC = 100352


def _sc_body(x_hbm, y_hbm, z_hbm, i_hbm, j_hbm, out_hbm,
             acc, x0, y0, z0, i0, j0, x1, y1, z1, i1, j1, sem0, sem1):
    cid = lax.axis_index("c")
    sid = lax.axis_index("s")
    wid = cid * NS + sid
    ebase = wid * EPW

    slots = ((x0, y0, z0, i0, j0, sem0), (x1, y1, z1, i1, j1, sem1))
    hbms = (x_hbm, y_hbm, z_hbm, i_hbm, j_hbm)

    def issue5(slot, k):
        base = ebase + k * CHUNK
        for hbm, buf in zip(hbms, slot[:5]):
            pltpu.async_copy(hbm.at[pl.ds(base, CHUNK)], buf, slot[5])

    def wait5(slot):
        for hbm, buf in zip(hbms, slot[:5]):
            pltpu.make_async_copy(hbm.at[pl.ds(0, CHUNK)], buf, slot[5]).wait()

    issue5(slots[0], 0)

    zero16 = jnp.zeros((16,), jnp.float32)

    @plsc.parallel_loop(0, ACC // 64, 1, unroll=4)
    def _zero(t):
        for q in range(4):
            acc[pl.ds(t * 64 + q * 16, 16)] = zero16

    def compute(slot):
        xb, yb, zb, ib, jb = slot[:5]

        @plsc.parallel_loop(0, GROUPS, 1, unroll=UNROLL)
        def _group(g):
            o = g * 16
            dx = xb[pl.ds(o, 16)]
            dy = yb[pl.ds(o, 16)]
            dz = zb[pl.ds(o, 16)]
            r2 = dx * dx + dy * dy + dz * dz
            inv = 1.0 / r2
            s6 = inv * inv * inv
            he = 2.0 * _EPS * (s6 * s6 - s6) - _HALF_SHIFT
            iv = ib[pl.ds(o, 16)]
            jv = jb[pl.ds(o, 16)]
            plsc.addupdate_scatter(acc, [iv], he)
            plsc.addupdate_scatter(acc, [jv], he)

    def _pair(t, _):
        issue5(slots[1], 2 * t + 1)
        wait5(slots[0])
        compute(slots[0])

        @pl.when(t < NPAIR - 1)
        def _():
            issue5(slots[0], 2 * t + 2)

        wait5(slots[1])
        compute(slots[1])
        return 0

    lax.fori_loop(0, NPAIR, _pair, 0)

    pltpu.sync_copy(acc, out_hbm.at[wid])


@functools.partial(jax.jit, static_argnames=())
def _sc_partials(xs, ys, zs, all_i, all_j):
    mesh = plsc.VectorSubcoreMesh(core_axis_name="c", subcore_axis_name="s",
                                  num_cores=NC, num_subcores=NS)
    return pl.kernel(
        _sc_body,
        out_type=jax.ShapeDtypeStruct((NW, ACC), jnp.float32),
        mesh=mesh,
        compiler_params=pltpu.CompilerParams(needs_layout_passes=False,
                                             use_tc_tiling_on_sc=False),
        scratch_types=[
            pltpu.VMEM((ACC,), jnp.float32),
            pltpu.VMEM((CHUNK,), jnp.float32),
            pltpu.VMEM((CHUNK,), jnp.float32),
            pltpu.VMEM((CHUNK,), jnp.float32),
            pltpu.VMEM((CHUNK,), jnp.int32),
            pltpu.VMEM((CHUNK,), jnp.int32),
            pltpu.VMEM((CHUNK,), jnp.float32),
            pltpu.VMEM((CHUNK,), jnp.float32),
            pltpu.VMEM((CHUNK,), jnp.float32),
            pltpu.VMEM((CHUNK,), jnp.int32),
            pltpu.VMEM((CHUNK,), jnp.int32),
            pltpu.SemaphoreType.DMA,
            pltpu.SemaphoreType.DMA,
        ],
    )(xs, ys, zs, all_i, all_j)


def _sum_body(p_ref, b_ref, o_ref):
    o_ref[...] = jnp.sum(p_ref[...], axis=0) + b_ref[...]


def _tc_sum(partials, bias):
    return pl.pallas_call(
        _sum_body,
        out_shape=jax.ShapeDtypeStruct((ACC,), jnp.float32),
    )(partials, bias)


def kernel(distances, all_i, all_j, n_nodes):
    xs = distances[:, 0]
    ys = distances[:, 1]
    zs = distances[:, 2]
    partials = _sc_partials(xs, ys, zs, all_i, all_j)
    bias = jnp.full((1,), 0.0, jnp.float32) + (
        jnp.asarray(n_nodes, jnp.float32) - float(N_NODES_C))
    summed = _tc_sum(partials, bias)
    return summed[:N_NODES_C].reshape(-1, 1)
